# baseline (speedup 1.0000x reference)
import functools

import jax
import jax.numpy as jnp
from jax import lax
from jax.experimental import pallas as pl
from jax.experimental.pallas import tpu as pltpu
from jax.experimental.pallas import tpu_sc as plsc

N = 10000
E = 320000
D_IN = 128
D_HID = 128
D_OUT = 64

NC = 2
NS = 16
NW = NC * NS

N_PAD = 10000
ROWS_PER_TILE = N_PAD // NS
E_PER_TILE = E // NW

CHUNK = 100
CHUNK128 = 100
CHUNK64 = 125

_MESH = plsc.VectorSubcoreMesh(core_axis_name="c", subcore_axis_name="s")
_SC_PARAMS = pltpu.CompilerParams(use_tc_tiling_on_sc=False)

NBUF = 2


def _make_agg_kernel(d, chunk):
    n_chunks = E_PER_TILE // chunk

    @functools.partial(
        pl.kernel,
        mesh=_MESH,
        out_type=jax.ShapeDtypeStruct((NC, N_PAD, d), jnp.float32),
        compiler_params=_SC_PARAMS,
        scratch_types=[
            pltpu.VMEM((n_chunks, chunk), jnp.int32),
            pltpu.VMEM((n_chunks, chunk), jnp.int32),
            [pltpu.VMEM((chunk, d), jnp.float32)] * NBUF,
            pltpu.VMEM_SHARED((N_PAD, d), jnp.float32),
            [pltpu.SemaphoreType.DMA] * NBUF,
            [pltpu.SemaphoreType.DMA] * NBUF,
        ],
    )
    def agg(z_hbm, src_hbm, dst_hbm, zeros_hbm, out_hbm,
            src_all, dst_all, rows, acc_sh, gsems, ssems):
        cid = lax.axis_index("c")
        sid = lax.axis_index("s")
        r0 = sid * ROWS_PER_TILE
        wid = sid * NC + cid
        c0 = wid * n_chunks
        pltpu.sync_copy(src_hbm.at[pl.ds(c0, n_chunks)], src_all)
        pltpu.sync_copy(dst_hbm.at[pl.ds(c0, n_chunks)], dst_all)
        pltpu.sync_copy(zeros_hbm.at[pl.ds(r0, ROWS_PER_TILE)],
                        acc_sh.at[pl.ds(r0, ROWS_PER_TILE)])
        plsc.subcore_barrier()

        for b in range(NBUF):
            pltpu.async_copy(z_hbm.at[src_all.at[b]], rows[b], gsems[b])

        def body(c_base, carry):
            for b in range(NBUF):
                c = c_base + b
                pltpu.make_async_copy(
                    z_hbm.at[src_all.at[c]], rows[b], gsems[b]).wait()
                pltpu.async_copy(
                    rows[b], acc_sh.at[dst_all.at[c]], ssems[b], add=True)
            for b in range(NBUF):
                nxt = c_base + b + NBUF

                @pl.when(nxt < n_chunks)
                def _():
                    pltpu.make_async_copy(
                        rows[b], acc_sh.at[dst_all.at[b]], ssems[b]).wait()
                    pltpu.async_copy(
                        z_hbm.at[src_all.at[nxt]], rows[b], gsems[b])

            return carry

        lax.fori_loop(0, n_chunks // NBUF, lambda i, car: body(i * NBUF, car), 0)
        for b in range(NBUF):
            pltpu.make_async_copy(
                rows[b], acc_sh.at[dst_all.at[b]], ssems[b]).wait()
        plsc.subcore_barrier()
        pltpu.sync_copy(acc_sh.at[pl.ds(r0, ROWS_PER_TILE)],
                        out_hbm.at[cid, pl.ds(r0, ROWS_PER_TILE)])

    return agg


def _make_deg_kernel():
    n_chunks = E_PER_TILE // CHUNK

    @functools.partial(
        pl.kernel,
        mesh=_MESH,
        out_type=jax.ShapeDtypeStruct((NC, N_PAD, 16), jnp.float32),
        compiler_params=_SC_PARAMS,
        scratch_types=[
            pltpu.VMEM((n_chunks, CHUNK), jnp.int32),
            pltpu.VMEM((CHUNK, 16), jnp.float32),
            pltpu.VMEM_SHARED((N_PAD, 16), jnp.float32),
            pltpu.SemaphoreType.DMA,
        ],
    )
    def deg(dst_hbm, zeros_hbm, out_hbm, dst_all, ones_v, acc_sh, sem):
        cid = lax.axis_index("c")
        sid = lax.axis_index("s")
        r0 = sid * ROWS_PER_TILE
        wid = sid * NC + cid
        c0 = wid * n_chunks
        pltpu.sync_copy(dst_hbm.at[pl.ds(c0, n_chunks)], dst_all)
        pltpu.sync_copy(zeros_hbm.at[pl.ds(r0, ROWS_PER_TILE)],
                        acc_sh.at[pl.ds(r0, ROWS_PER_TILE)])
        for j in range(CHUNK):
            ones_v[j, :] = jnp.ones((16,), jnp.float32)
        plsc.subcore_barrier()

        def body(c_base, carry):
            for b in range(NBUF):
                pltpu.async_copy(
                    ones_v, acc_sh.at[dst_all.at[c_base + b]], sem, add=True)
            for b in range(NBUF):
                pltpu.make_async_copy(
                    ones_v, acc_sh.at[dst_all.at[c_base + b]], sem).wait()
            return carry

        lax.fori_loop(0, n_chunks // NBUF, lambda i, car: body(i * NBUF, car), 0)
        plsc.subcore_barrier()
        pltpu.sync_copy(acc_sh.at[pl.ds(r0, ROWS_PER_TILE)],
                        out_hbm.at[cid, pl.ds(r0, ROWS_PER_TILE)])

    return deg


_R = 1000


def _tc1(x, w1, degp):
    def body(x_ref, w_ref, p_ref, z_ref, dis_ref):
        deg = 1.0 + p_ref[0] + p_ref[1]
        disv = lax.rsqrt(deg)
        dis_ref[...] = disv
        mm = jnp.dot(x_ref[...], w_ref[...], preferred_element_type=jnp.float32)
        z_ref[...] = disv[:, 0:1] * mm

    return pl.pallas_call(
        body,
        grid=(N // _R,),
        in_specs=[
            pl.BlockSpec((_R, D_IN), lambda i: (i, 0)),
            pl.BlockSpec((D_IN, D_HID), lambda i: (0, 0)),
            pl.BlockSpec((NC, _R, 16), lambda i: (0, i, 0)),
        ],
        out_specs=[
            pl.BlockSpec((_R, D_HID), lambda i: (i, 0)),
            pl.BlockSpec((_R, 16), lambda i: (i, 0)),
        ],
        out_shape=[
            jax.ShapeDtypeStruct((N, D_HID), jnp.float32),
            jax.ShapeDtypeStruct((N, 16), jnp.float32),
        ],
    )(x, w1, degp)


def _tc2(aggp, z1, dis, b1, w2):
    def body(a_ref, z_ref, dis_ref, b1_ref, w_ref, out_ref):
        pre = dis_ref[:, 0:1] * (a_ref[0] + a_ref[1] + z_ref[...]) + b1_ref[...]
        h = 1.0 / (1.0 + jnp.exp(-pre))
        mm = jnp.dot(h, w_ref[...], preferred_element_type=jnp.float32)
        out_ref[...] = dis_ref[:, 0:1] * mm

    return pl.pallas_call(
        body,
        grid=(N // _R,),
        in_specs=[
            pl.BlockSpec((NC, _R, D_HID), lambda i: (0, i, 0)),
            pl.BlockSpec((_R, D_HID), lambda i: (i, 0)),
            pl.BlockSpec((_R, 16), lambda i: (i, 0)),
            pl.BlockSpec((1, D_HID), lambda i: (0, 0)),
            pl.BlockSpec((D_HID, D_OUT), lambda i: (0, 0)),
        ],
        out_specs=pl.BlockSpec((_R, D_OUT), lambda i: (i, 0)),
        out_shape=jax.ShapeDtypeStruct((N, D_OUT), jnp.float32),
    )(aggp, z1, dis, b1, w2)


def _tc3(aggp, z2, dis, b2):
    def body(a_ref, z_ref, dis_ref, b2_ref, out_ref):
        out_ref[...] = (
            dis_ref[:, 0:1] * (a_ref[0] + a_ref[1] + z_ref[...]) + b2_ref[...]
        )

    return pl.pallas_call(
        body,
        grid=(N // _R,),
        in_specs=[
            pl.BlockSpec((NC, _R, D_OUT), lambda i: (0, i, 0)),
            pl.BlockSpec((_R, D_OUT), lambda i: (i, 0)),
            pl.BlockSpec((_R, 16), lambda i: (i, 0)),
            pl.BlockSpec((1, D_OUT), lambda i: (0, 0)),
        ],
        out_specs=pl.BlockSpec((_R, D_OUT), lambda i: (i, 0)),
        out_shape=jax.ShapeDtypeStruct((N, D_OUT), jnp.float32),
    )(aggp, z2, dis, b2)


_deg_kernel = _make_deg_kernel()
_agg128 = _make_agg_kernel(D_HID, CHUNK128)
_agg64 = _make_agg_kernel(D_OUT, CHUNK64)


def kernel(x, edges, W1, b1, W2, b2):
    edges = edges.astype(jnp.int32)
    src = edges[0]
    dst = edges[1]
    zeros16 = jnp.zeros((N_PAD, 16), jnp.float32)
    zeros128 = jnp.zeros((N_PAD, D_HID), jnp.float32)
    zeros64 = jnp.zeros((N_PAD, D_OUT), jnp.float32)
    b1f = b1.reshape(1, D_HID)
    b2f = b2.reshape(1, D_OUT)

    degp = _deg_kernel(dst.reshape(E // CHUNK, CHUNK), zeros16)
    z1, dis = _tc1(x, W1, degp)
    agg1 = _agg128(z1, src.reshape(E // CHUNK128, CHUNK128),
                   dst.reshape(E // CHUNK128, CHUNK128), zeros128)
    z2 = _tc2(agg1, z1, dis, b1f, W2)
    agg2 = _agg64(z2, src.reshape(E // CHUNK64, CHUNK64),
                  dst.reshape(E // CHUNK64, CHUNK64), zeros64)
    out = _tc3(agg2, z2, dis, b2f)
    return out


# speedup vs baseline: 1.1777x; 1.1777x over previous
import functools

import jax
import jax.numpy as jnp
from jax import lax
from jax.experimental import pallas as pl
from jax.experimental.pallas import tpu as pltpu
from jax.experimental.pallas import tpu_sc as plsc

N = 10000
E = 320000
D_IN = 128
D_HID = 128
D_OUT = 64

NC = 2
NS = 16
NW = NC * NS

N_PAD = 10000
ROWS_PER_TILE = N_PAD // NS
E_PER_TILE = E // NW

CHUNK = 100
CHUNK128 = 100
CHUNK64 = 125

_MESH = plsc.VectorSubcoreMesh(core_axis_name="c", subcore_axis_name="s")
_SC_PARAMS = pltpu.CompilerParams(use_tc_tiling_on_sc=False)

NBUF = 2


def _make_agg_kernel(d, chunk):
    n_chunks = E_PER_TILE // chunk

    @functools.partial(
        pl.kernel,
        mesh=_MESH,
        out_type=jax.ShapeDtypeStruct((NC, N_PAD, d), jnp.float32),
        compiler_params=_SC_PARAMS,
        scratch_types=[
            pltpu.VMEM((n_chunks, chunk), jnp.int32),
            pltpu.VMEM((n_chunks, chunk), jnp.int32),
            [pltpu.VMEM((chunk, d), jnp.float32)] * NBUF,
            pltpu.VMEM_SHARED((N_PAD, d), jnp.float32),
            [pltpu.SemaphoreType.DMA] * NBUF,
            [pltpu.SemaphoreType.DMA] * NBUF,
        ],
    )
    def agg(z_hbm, src_hbm, dst_hbm, zeros_hbm, out_hbm,
            src_all, dst_all, rows, acc_sh, gsems, ssems):
        cid = lax.axis_index("c")
        sid = lax.axis_index("s")
        r0 = sid * ROWS_PER_TILE
        wid = sid * NC + cid
        c0 = wid * n_chunks
        pltpu.sync_copy(src_hbm.at[pl.ds(c0, n_chunks)], src_all)
        pltpu.sync_copy(dst_hbm.at[pl.ds(c0, n_chunks)], dst_all)
        pltpu.sync_copy(zeros_hbm.at[pl.ds(r0, ROWS_PER_TILE)],
                        acc_sh.at[pl.ds(r0, ROWS_PER_TILE)])
        plsc.subcore_barrier()

        for b in range(NBUF):
            pltpu.async_copy(z_hbm.at[src_all.at[b]], rows[b], gsems[b])

        def body(c_base, carry):
            for b in range(NBUF):
                c = c_base + b
                pltpu.make_async_copy(
                    z_hbm.at[src_all.at[c]], rows[b], gsems[b]).wait()
                pltpu.sync_copy(rows[b], acc_sh.at[dst_all.at[c]], add=True)
                nxt = c + NBUF

                @pl.when(nxt < n_chunks)
                def _():
                    pltpu.async_copy(
                        z_hbm.at[src_all.at[nxt]], rows[b], gsems[b])

            return carry

        lax.fori_loop(0, n_chunks // NBUF, lambda i, car: body(i * NBUF, car), 0)
        plsc.subcore_barrier()
        pltpu.sync_copy(acc_sh.at[pl.ds(r0, ROWS_PER_TILE)],
                        out_hbm.at[cid, pl.ds(r0, ROWS_PER_TILE)])

    return agg


def _make_deg_kernel():
    n_chunks = E_PER_TILE // CHUNK

    @functools.partial(
        pl.kernel,
        mesh=_MESH,
        out_type=jax.ShapeDtypeStruct((NC, N_PAD, 16), jnp.float32),
        compiler_params=_SC_PARAMS,
        scratch_types=[
            pltpu.VMEM((n_chunks, CHUNK), jnp.int32),
            pltpu.VMEM((CHUNK, 16), jnp.float32),
            pltpu.VMEM_SHARED((N_PAD, 16), jnp.float32),
            pltpu.SemaphoreType.DMA,
        ],
    )
    def deg(dst_hbm, zeros_hbm, out_hbm, dst_all, ones_v, acc_sh, sem):
        cid = lax.axis_index("c")
        sid = lax.axis_index("s")
        r0 = sid * ROWS_PER_TILE
        wid = sid * NC + cid
        c0 = wid * n_chunks
        pltpu.sync_copy(dst_hbm.at[pl.ds(c0, n_chunks)], dst_all)
        pltpu.sync_copy(zeros_hbm.at[pl.ds(r0, ROWS_PER_TILE)],
                        acc_sh.at[pl.ds(r0, ROWS_PER_TILE)])
        for j in range(CHUNK):
            ones_v[j, :] = jnp.ones((16,), jnp.float32)
        plsc.subcore_barrier()

        def body(c_base, carry):
            for b in range(NBUF):
                pltpu.async_copy(
                    ones_v, acc_sh.at[dst_all.at[c_base + b]], sem, add=True)
            for b in range(NBUF):
                pltpu.make_async_copy(
                    ones_v, acc_sh.at[dst_all.at[c_base + b]], sem).wait()
            return carry

        lax.fori_loop(0, n_chunks // NBUF, lambda i, car: body(i * NBUF, car), 0)
        plsc.subcore_barrier()
        pltpu.sync_copy(acc_sh.at[pl.ds(r0, ROWS_PER_TILE)],
                        out_hbm.at[cid, pl.ds(r0, ROWS_PER_TILE)])

    return deg


_R = 1000


def _tc1(x, w1, degp):
    def body(x_ref, w_ref, p_ref, z_ref, dis_ref):
        deg = 1.0 + p_ref[0] + p_ref[1]
        disv = lax.rsqrt(deg)
        dis_ref[...] = disv
        mm = jnp.dot(x_ref[...], w_ref[...], preferred_element_type=jnp.float32)
        z_ref[...] = disv[:, 0:1] * mm

    return pl.pallas_call(
        body,
        grid=(N // _R,),
        in_specs=[
            pl.BlockSpec((_R, D_IN), lambda i: (i, 0)),
            pl.BlockSpec((D_IN, D_HID), lambda i: (0, 0)),
            pl.BlockSpec((NC, _R, 16), lambda i: (0, i, 0)),
        ],
        out_specs=[
            pl.BlockSpec((_R, D_HID), lambda i: (i, 0)),
            pl.BlockSpec((_R, 16), lambda i: (i, 0)),
        ],
        out_shape=[
            jax.ShapeDtypeStruct((N, D_HID), jnp.float32),
            jax.ShapeDtypeStruct((N, 16), jnp.float32),
        ],
    )(x, w1, degp)


def _tc2(aggp, z1, dis, b1, w2):
    def body(a_ref, z_ref, dis_ref, b1_ref, w_ref, out_ref):
        pre = dis_ref[:, 0:1] * (a_ref[0] + a_ref[1] + z_ref[...]) + b1_ref[...]
        h = 1.0 / (1.0 + jnp.exp(-pre))
        mm = jnp.dot(h, w_ref[...], preferred_element_type=jnp.float32)
        out_ref[...] = dis_ref[:, 0:1] * mm

    return pl.pallas_call(
        body,
        grid=(N // _R,),
        in_specs=[
            pl.BlockSpec((NC, _R, D_HID), lambda i: (0, i, 0)),
            pl.BlockSpec((_R, D_HID), lambda i: (i, 0)),
            pl.BlockSpec((_R, 16), lambda i: (i, 0)),
            pl.BlockSpec((1, D_HID), lambda i: (0, 0)),
            pl.BlockSpec((D_HID, D_OUT), lambda i: (0, 0)),
        ],
        out_specs=pl.BlockSpec((_R, D_OUT), lambda i: (i, 0)),
        out_shape=jax.ShapeDtypeStruct((N, D_OUT), jnp.float32),
    )(aggp, z1, dis, b1, w2)


def _tc3(aggp, z2, dis, b2):
    def body(a_ref, z_ref, dis_ref, b2_ref, out_ref):
        out_ref[...] = (
            dis_ref[:, 0:1] * (a_ref[0] + a_ref[1] + z_ref[...]) + b2_ref[...]
        )

    return pl.pallas_call(
        body,
        grid=(N // _R,),
        in_specs=[
            pl.BlockSpec((NC, _R, D_OUT), lambda i: (0, i, 0)),
            pl.BlockSpec((_R, D_OUT), lambda i: (i, 0)),
            pl.BlockSpec((_R, 16), lambda i: (i, 0)),
            pl.BlockSpec((1, D_OUT), lambda i: (0, 0)),
        ],
        out_specs=pl.BlockSpec((_R, D_OUT), lambda i: (i, 0)),
        out_shape=jax.ShapeDtypeStruct((N, D_OUT), jnp.float32),
    )(aggp, z2, dis, b2)


_deg_kernel = _make_deg_kernel()
_agg128 = _make_agg_kernel(D_HID, CHUNK128)
_agg64 = _make_agg_kernel(D_OUT, CHUNK64)


def kernel(x, edges, W1, b1, W2, b2):
    edges = edges.astype(jnp.int32)
    src = edges[0]
    dst = edges[1]
    zeros16 = jnp.zeros((N_PAD, 16), jnp.float32)
    zeros128 = jnp.zeros((N_PAD, D_HID), jnp.float32)
    zeros64 = jnp.zeros((N_PAD, D_OUT), jnp.float32)
    b1f = b1.reshape(1, D_HID)
    b2f = b2.reshape(1, D_OUT)

    degp = _deg_kernel(dst.reshape(E // CHUNK, CHUNK), zeros16)
    z1, dis = _tc1(x, W1, degp)
    agg1 = _agg128(z1, src.reshape(E // CHUNK128, CHUNK128),
                   dst.reshape(E // CHUNK128, CHUNK128), zeros128)
    z2 = _tc2(agg1, z1, dis, b1f, W2)
    agg2 = _agg64(z2, src.reshape(E // CHUNK64, CHUNK64),
                  dst.reshape(E // CHUNK64, CHUNK64), zeros64)
    out = _tc3(agg2, z2, dis, b2f)
    return out


# speedup vs baseline: 1.1785x; 1.0007x over previous
import functools

import jax
import jax.numpy as jnp
from jax import lax
from jax.experimental import pallas as pl
from jax.experimental.pallas import tpu as pltpu
from jax.experimental.pallas import tpu_sc as plsc

N = 10000
E = 320000
D_IN = 128
D_HID = 128
D_OUT = 64

NC = 2
NS = 16
NW = NC * NS

N_PAD = 10000
ROWS_PER_TILE = N_PAD // NS
E_PER_TILE = E // NW

CHUNK = 100
N_CHUNKS_ALL = E // CHUNK

_MESH = plsc.VectorSubcoreMesh(core_axis_name="c", subcore_axis_name="s")
_SC_PARAMS = pltpu.CompilerParams(use_tc_tiling_on_sc=False)

NBUF = 2


def _make_agg_kernel(d, chunk):
    n_chunks = E_PER_TILE // chunk

    @functools.partial(
        pl.kernel,
        mesh=_MESH,
        out_type=jax.ShapeDtypeStruct((NC, N_PAD, d), jnp.float32),
        compiler_params=_SC_PARAMS,
        scratch_types=[
            pltpu.VMEM((n_chunks, chunk), jnp.int32),
            pltpu.VMEM((n_chunks, chunk), jnp.int32),
            [pltpu.VMEM((chunk, d), jnp.float32)] * NBUF,
            pltpu.VMEM_SHARED((N_PAD, d), jnp.float32),
            [pltpu.SemaphoreType.DMA] * NBUF,
            [pltpu.SemaphoreType.DMA] * NBUF,
        ],
    )
    def agg(z_hbm, edges_hbm, zeros_hbm, out_hbm,
            src_all, dst_all, rows, acc_sh, gsems, ssems):
        cid = lax.axis_index("c")
        sid = lax.axis_index("s")
        r0 = sid * ROWS_PER_TILE
        wid = sid * NC + cid
        c0 = wid * n_chunks
        pltpu.sync_copy(edges_hbm.at[0, pl.ds(c0, n_chunks)], src_all)
        pltpu.sync_copy(edges_hbm.at[1, pl.ds(c0, n_chunks)], dst_all)
        pltpu.sync_copy(zeros_hbm.at[pl.ds(r0, ROWS_PER_TILE)],
                        acc_sh.at[pl.ds(r0, ROWS_PER_TILE)])
        plsc.subcore_barrier()

        for b in range(NBUF):
            pltpu.async_copy(z_hbm.at[src_all.at[b]], rows[b], gsems[b])

        def body(c_base, carry):
            for b in range(NBUF):
                c = c_base + b
                pltpu.make_async_copy(
                    z_hbm.at[src_all.at[c]], rows[b], gsems[b]).wait()
                pltpu.sync_copy(rows[b], acc_sh.at[dst_all.at[c]], add=True)
                nxt = c + NBUF

                @pl.when(nxt < n_chunks)
                def _():
                    pltpu.async_copy(
                        z_hbm.at[src_all.at[nxt]], rows[b], gsems[b])

            return carry

        lax.fori_loop(0, n_chunks // NBUF, lambda i, car: body(i * NBUF, car), 0)
        plsc.subcore_barrier()
        pltpu.sync_copy(acc_sh.at[pl.ds(r0, ROWS_PER_TILE)],
                        out_hbm.at[cid, pl.ds(r0, ROWS_PER_TILE)])

    return agg


def _make_deg_kernel():
    n_chunks = E_PER_TILE // CHUNK

    @functools.partial(
        pl.kernel,
        mesh=_MESH,
        out_type=jax.ShapeDtypeStruct((NC, N_PAD, 16), jnp.float32),
        compiler_params=_SC_PARAMS,
        scratch_types=[
            pltpu.VMEM((n_chunks, CHUNK), jnp.int32),
            pltpu.VMEM((CHUNK, 16), jnp.float32),
            pltpu.VMEM_SHARED((N_PAD, 16), jnp.float32),
            pltpu.SemaphoreType.DMA,
        ],
    )
    def deg(edges_hbm, zeros_hbm, out_hbm, dst_all, ones_v, acc_sh, sem):
        cid = lax.axis_index("c")
        sid = lax.axis_index("s")
        r0 = sid * ROWS_PER_TILE
        wid = sid * NC + cid
        c0 = wid * n_chunks
        pltpu.sync_copy(edges_hbm.at[1, pl.ds(c0, n_chunks)], dst_all)
        pltpu.sync_copy(zeros_hbm.at[pl.ds(r0, ROWS_PER_TILE)],
                        acc_sh.at[pl.ds(r0, ROWS_PER_TILE)])
        for j in range(CHUNK):
            ones_v[j, :] = jnp.ones((16,), jnp.float32)
        plsc.subcore_barrier()

        def body(c_base, carry):
            for b in range(NBUF):
                pltpu.async_copy(
                    ones_v, acc_sh.at[dst_all.at[c_base + b]], sem, add=True)
            for b in range(NBUF):
                pltpu.make_async_copy(
                    ones_v, acc_sh.at[dst_all.at[c_base + b]], sem).wait()
            return carry

        lax.fori_loop(0, n_chunks // NBUF, lambda i, car: body(i * NBUF, car), 0)
        plsc.subcore_barrier()
        pltpu.sync_copy(acc_sh.at[pl.ds(r0, ROWS_PER_TILE)],
                        out_hbm.at[cid, pl.ds(r0, ROWS_PER_TILE)])

    return deg


_R = 1000


def _tc1(x, w1, degp):
    def body(x_ref, w_ref, p_ref, z_ref, dis_ref):
        deg = 1.0 + p_ref[0] + p_ref[1]
        disv = lax.rsqrt(deg)
        dis_ref[...] = disv
        mm = jnp.dot(x_ref[...], w_ref[...], preferred_element_type=jnp.float32)
        z_ref[...] = disv[:, 0:1] * mm

    return pl.pallas_call(
        body,
        grid=(N // _R,),
        in_specs=[
            pl.BlockSpec((_R, D_IN), lambda i: (i, 0)),
            pl.BlockSpec((D_IN, D_HID), lambda i: (0, 0)),
            pl.BlockSpec((NC, _R, 16), lambda i: (0, i, 0)),
        ],
        out_specs=[
            pl.BlockSpec((_R, D_HID), lambda i: (i, 0)),
            pl.BlockSpec((_R, 16), lambda i: (i, 0)),
        ],
        out_shape=[
            jax.ShapeDtypeStruct((N, D_HID), jnp.float32),
            jax.ShapeDtypeStruct((N, 16), jnp.float32),
        ],
    )(x, w1, degp)


def _tc2(aggp, z1, dis, b1, w2):
    def body(a_ref, z_ref, dis_ref, b1_ref, w_ref, out_ref):
        pre = dis_ref[:, 0:1] * (a_ref[0] + a_ref[1] + z_ref[...]) + b1_ref[...]
        h = 1.0 / (1.0 + jnp.exp(-pre))
        mm = jnp.dot(h, w_ref[...], preferred_element_type=jnp.float32)
        out_ref[...] = dis_ref[:, 0:1] * mm

    return pl.pallas_call(
        body,
        grid=(N // _R,),
        in_specs=[
            pl.BlockSpec((NC, _R, D_HID), lambda i: (0, i, 0)),
            pl.BlockSpec((_R, D_HID), lambda i: (i, 0)),
            pl.BlockSpec((_R, 16), lambda i: (i, 0)),
            pl.BlockSpec((1, D_HID), lambda i: (0, 0)),
            pl.BlockSpec((D_HID, D_OUT), lambda i: (0, 0)),
        ],
        out_specs=pl.BlockSpec((_R, D_OUT), lambda i: (i, 0)),
        out_shape=jax.ShapeDtypeStruct((N, D_OUT), jnp.float32),
    )(aggp, z1, dis, b1, w2)


def _tc3(aggp, z2, dis, b2):
    def body(a_ref, z_ref, dis_ref, b2_ref, out_ref):
        out_ref[...] = (
            dis_ref[:, 0:1] * (a_ref[0] + a_ref[1] + z_ref[...]) + b2_ref[...]
        )

    return pl.pallas_call(
        body,
        grid=(N // _R,),
        in_specs=[
            pl.BlockSpec((NC, _R, D_OUT), lambda i: (0, i, 0)),
            pl.BlockSpec((_R, D_OUT), lambda i: (i, 0)),
            pl.BlockSpec((_R, 16), lambda i: (i, 0)),
            pl.BlockSpec((1, D_OUT), lambda i: (0, 0)),
        ],
        out_specs=pl.BlockSpec((_R, D_OUT), lambda i: (i, 0)),
        out_shape=jax.ShapeDtypeStruct((N, D_OUT), jnp.float32),
    )(aggp, z2, dis, b2)


_deg_kernel = _make_deg_kernel()
_agg128 = _make_agg_kernel(D_HID, CHUNK)
_agg64 = _make_agg_kernel(D_OUT, CHUNK)


def kernel(x, edges, W1, b1, W2, b2):
    edges3 = edges.astype(jnp.int32).reshape(2, N_CHUNKS_ALL, CHUNK)
    zeros16 = jnp.zeros((N_PAD, 16), jnp.float32)
    zeros128 = jnp.zeros((N_PAD, D_HID), jnp.float32)
    zeros64 = jnp.zeros((N_PAD, D_OUT), jnp.float32)
    b1f = b1.reshape(1, D_HID)
    b2f = b2.reshape(1, D_OUT)

    degp = _deg_kernel(edges3, zeros16)
    z1, dis = _tc1(x, W1, degp)
    agg1 = _agg128(z1, edges3, zeros128)
    z2 = _tc2(agg1, z1, dis, b1f, W2)
    agg2 = _agg64(z2, edges3, zeros64)
    out = _tc3(agg2, z2, dis, b2f)
    return out


# speedup vs baseline: 1.2116x; 1.0281x over previous
import functools

import jax
import jax.numpy as jnp
from jax import lax
from jax.experimental import pallas as pl
from jax.experimental.pallas import tpu as pltpu
from jax.experimental.pallas import tpu_sc as plsc

N = 10000
E = 320000
D_IN = 128
D_HID = 128
D_OUT = 64

NC = 2
NS = 16
NW = NC * NS

N_PAD = 10000
ROWS_PER_TILE = N_PAD // NS
E_PER_TILE = E // NW

CHUNK = 100
N_CHUNKS_ALL = E // CHUNK

_MESH = plsc.VectorSubcoreMesh(core_axis_name="c", subcore_axis_name="s")
_SC_PARAMS = pltpu.CompilerParams(use_tc_tiling_on_sc=False)

NBUF = 2


def _make_agg_kernel(d, chunk):
    n_chunks = E_PER_TILE // chunk

    @functools.partial(
        pl.kernel,
        mesh=_MESH,
        out_type=jax.ShapeDtypeStruct((NC, N_PAD, d), jnp.float32),
        compiler_params=_SC_PARAMS,
        scratch_types=[
            pltpu.VMEM((n_chunks, chunk), jnp.int32),
            pltpu.VMEM((n_chunks, chunk), jnp.int32),
            [pltpu.VMEM((chunk, d), jnp.float32)] * NBUF,
            pltpu.VMEM_SHARED((N_PAD, d), jnp.float32),
            [pltpu.SemaphoreType.DMA] * NBUF,
        ],
    )
    def agg(z_hbm, edges_hbm, out_hbm,
            src_all, dst_all, rows, acc_sh, gsems):
        cid = lax.axis_index("c")
        sid = lax.axis_index("s")
        r0 = sid * ROWS_PER_TILE
        wid = sid * NC + cid
        c0 = wid * n_chunks
        pltpu.sync_copy(edges_hbm.at[0, pl.ds(c0, n_chunks)], src_all)
        pltpu.sync_copy(edges_hbm.at[1, pl.ds(c0, n_chunks)], dst_all)
        zero16 = jnp.zeros((16,), jnp.float32)

        def zrow(j, car):
            for k in range(d // 16):
                rows[0][j, pl.ds(k * 16, 16)] = zero16
            return car

        lax.fori_loop(0, chunk, zrow, 0)
        full, rem = divmod(ROWS_PER_TILE, chunk)
        for t in range(full):
            pltpu.sync_copy(rows[0], acc_sh.at[pl.ds(r0 + t * chunk, chunk)])
        if rem:
            pltpu.sync_copy(rows[0].at[pl.ds(0, rem)],
                            acc_sh.at[pl.ds(r0 + full * chunk, rem)])
        plsc.subcore_barrier()

        for b in range(NBUF):
            pltpu.async_copy(z_hbm.at[src_all.at[b]], rows[b], gsems[b])

        def body(c_base, carry):
            for b in range(NBUF):
                c = c_base + b
                pltpu.make_async_copy(
                    z_hbm.at[src_all.at[c]], rows[b], gsems[b]).wait()
                pltpu.sync_copy(rows[b], acc_sh.at[dst_all.at[c]], add=True)
                pltpu.async_copy(
                    z_hbm.at[src_all.at[c + NBUF]], rows[b], gsems[b])
            return carry

        lax.fori_loop(0, (n_chunks - NBUF) // NBUF,
                      lambda i, car: body(i * NBUF, car), 0)
        for b in range(NBUF):
            c = n_chunks - NBUF + b
            pltpu.make_async_copy(
                z_hbm.at[src_all.at[c]], rows[b], gsems[b]).wait()
            pltpu.sync_copy(rows[b], acc_sh.at[dst_all.at[c]], add=True)
        plsc.subcore_barrier()
        pltpu.sync_copy(acc_sh.at[pl.ds(r0, ROWS_PER_TILE)],
                        out_hbm.at[cid, pl.ds(r0, ROWS_PER_TILE)])

    return agg


def _make_deg_kernel():
    n_chunks = E_PER_TILE // CHUNK

    @functools.partial(
        pl.kernel,
        mesh=_MESH,
        out_type=jax.ShapeDtypeStruct((NC, N_PAD, 16), jnp.float32),
        compiler_params=_SC_PARAMS,
        scratch_types=[
            pltpu.VMEM((n_chunks, CHUNK), jnp.int32),
            pltpu.VMEM((CHUNK, 16), jnp.float32),
            pltpu.VMEM_SHARED((N_PAD, 16), jnp.float32),
            pltpu.SemaphoreType.DMA,
        ],
    )
    def deg(edges_hbm, out_hbm, dst_all, ones_v, acc_sh, sem):
        cid = lax.axis_index("c")
        sid = lax.axis_index("s")
        r0 = sid * ROWS_PER_TILE
        wid = sid * NC + cid
        c0 = wid * n_chunks
        pltpu.sync_copy(edges_hbm.at[1, pl.ds(c0, n_chunks)], dst_all)
        zero16 = jnp.zeros((16,), jnp.float32)

        def zrow(j, car):
            ones_v[j, :] = zero16
            return car

        lax.fori_loop(0, CHUNK, zrow, 0)
        full, rem = divmod(ROWS_PER_TILE, CHUNK)
        for t in range(full):
            pltpu.sync_copy(ones_v, acc_sh.at[pl.ds(r0 + t * CHUNK, CHUNK)])
        if rem:
            pltpu.sync_copy(ones_v.at[pl.ds(0, rem)],
                            acc_sh.at[pl.ds(r0 + full * CHUNK, rem)])

        def orow(j, car):
            ones_v[j, :] = jnp.ones((16,), jnp.float32)
            return car

        lax.fori_loop(0, CHUNK, orow, 0)
        plsc.subcore_barrier()

        def body(c_base, carry):
            for b in range(NBUF):
                pltpu.async_copy(
                    ones_v, acc_sh.at[dst_all.at[c_base + b]], sem, add=True)
            for b in range(NBUF):
                pltpu.make_async_copy(
                    ones_v, acc_sh.at[dst_all.at[c_base + b]], sem).wait()
            return carry

        lax.fori_loop(0, n_chunks // NBUF, lambda i, car: body(i * NBUF, car), 0)
        plsc.subcore_barrier()
        pltpu.sync_copy(acc_sh.at[pl.ds(r0, ROWS_PER_TILE)],
                        out_hbm.at[cid, pl.ds(r0, ROWS_PER_TILE)])

    return deg


_R = 1000


def _tc_mm(x, w1):
    def body(x_ref, w_ref, u_ref):
        u_ref[...] = jnp.dot(
            x_ref[...], w_ref[...], preferred_element_type=jnp.float32)

    return pl.pallas_call(
        body,
        grid=(N // _R,),
        in_specs=[
            pl.BlockSpec((_R, D_IN), lambda i: (i, 0)),
            pl.BlockSpec((D_IN, D_HID), lambda i: (0, 0)),
        ],
        out_specs=pl.BlockSpec((_R, D_HID), lambda i: (i, 0)),
        out_shape=jax.ShapeDtypeStruct((N, D_HID), jnp.float32),
    )(x, w1)


def _tc_scale(u1, degp):
    def body(u_ref, p_ref, z_ref, dis_ref):
        deg = 1.0 + p_ref[0] + p_ref[1]
        disv = lax.rsqrt(deg)
        dis_ref[...] = disv
        z_ref[...] = disv[:, 0:1] * u_ref[...]

    return pl.pallas_call(
        body,
        grid=(N // _R,),
        in_specs=[
            pl.BlockSpec((_R, D_HID), lambda i: (i, 0)),
            pl.BlockSpec((NC, _R, 16), lambda i: (0, i, 0)),
        ],
        out_specs=[
            pl.BlockSpec((_R, D_HID), lambda i: (i, 0)),
            pl.BlockSpec((_R, 16), lambda i: (i, 0)),
        ],
        out_shape=[
            jax.ShapeDtypeStruct((N, D_HID), jnp.float32),
            jax.ShapeDtypeStruct((N, 16), jnp.float32),
        ],
    )(u1, degp)


def _tc2(aggp, z1, dis, b1, w2):
    def body(a_ref, z_ref, dis_ref, b1_ref, w_ref, out_ref):
        pre = dis_ref[:, 0:1] * (a_ref[0] + a_ref[1] + z_ref[...]) + b1_ref[...]
        h = 1.0 / (1.0 + jnp.exp(-pre))
        mm = jnp.dot(h, w_ref[...], preferred_element_type=jnp.float32)
        out_ref[...] = dis_ref[:, 0:1] * mm

    return pl.pallas_call(
        body,
        grid=(N // _R,),
        in_specs=[
            pl.BlockSpec((NC, _R, D_HID), lambda i: (0, i, 0)),
            pl.BlockSpec((_R, D_HID), lambda i: (i, 0)),
            pl.BlockSpec((_R, 16), lambda i: (i, 0)),
            pl.BlockSpec((1, D_HID), lambda i: (0, 0)),
            pl.BlockSpec((D_HID, D_OUT), lambda i: (0, 0)),
        ],
        out_specs=pl.BlockSpec((_R, D_OUT), lambda i: (i, 0)),
        out_shape=jax.ShapeDtypeStruct((N, D_OUT), jnp.float32),
    )(aggp, z1, dis, b1, w2)


def _tc3(aggp, z2, dis, b2):
    def body(a_ref, z_ref, dis_ref, b2_ref, out_ref):
        out_ref[...] = (
            dis_ref[:, 0:1] * (a_ref[0] + a_ref[1] + z_ref[...]) + b2_ref[...]
        )

    return pl.pallas_call(
        body,
        grid=(N // _R,),
        in_specs=[
            pl.BlockSpec((NC, _R, D_OUT), lambda i: (0, i, 0)),
            pl.BlockSpec((_R, D_OUT), lambda i: (i, 0)),
            pl.BlockSpec((_R, 16), lambda i: (i, 0)),
            pl.BlockSpec((1, D_OUT), lambda i: (0, 0)),
        ],
        out_specs=pl.BlockSpec((_R, D_OUT), lambda i: (i, 0)),
        out_shape=jax.ShapeDtypeStruct((N, D_OUT), jnp.float32),
    )(aggp, z2, dis, b2)


_deg_kernel = _make_deg_kernel()
_agg128 = _make_agg_kernel(D_HID, CHUNK)
_agg64 = _make_agg_kernel(D_OUT, CHUNK)


def kernel(x, edges, W1, b1, W2, b2):
    edges3 = edges.astype(jnp.int32).reshape(2, N_CHUNKS_ALL, CHUNK)
    b1f = b1.reshape(1, D_HID)
    b2f = b2.reshape(1, D_OUT)

    degp = _deg_kernel(edges3)
    u1 = _tc_mm(x, W1)
    z1, dis = _tc_scale(u1, degp)
    agg1 = _agg128(z1, edges3)
    z2 = _tc2(agg1, z1, dis, b1f, W2)
    agg2 = _agg64(z2, edges3)
    out = _tc3(agg2, z2, dis, b2f)
    return out


# speedup vs baseline: 1.2958x; 1.0695x over previous
import functools

import jax
import jax.numpy as jnp
from jax import lax
from jax.experimental import pallas as pl
from jax.experimental.pallas import tpu as pltpu
from jax.experimental.pallas import tpu_sc as plsc

N = 10000
E = 320000
D_IN = 128
D_HID = 128
D_OUT = 64

NC = 2
NS = 16
NW = NC * NS

N_PAD = 10000
ROWS_PER_TILE = N_PAD // NS
E_PER_TILE = E // NW

CHUNK = 128
N_CROWS = E // CHUNK
BASE = N_CROWS // NW
EXTRA = N_CROWS - BASE * NW
QUADS = BASE // 4
assert BASE - QUADS * 4 == 2

_MESH = plsc.VectorSubcoreMesh(core_axis_name="c", subcore_axis_name="s")
_SC_PARAMS = pltpu.CompilerParams(use_tc_tiling_on_sc=False)

NBUF = 2
NIDX = 4


def _make_agg_kernel(d):
    @functools.partial(
        pl.kernel,
        mesh=_MESH,
        out_type=jax.ShapeDtypeStruct((NC, N_PAD, d), jnp.float32),
        compiler_params=_SC_PARAMS,
        scratch_types=[
            [pltpu.VMEM((1, CHUNK), jnp.int32)] * NIDX,
            [pltpu.VMEM((1, CHUNK), jnp.int32)] * NIDX,
            [pltpu.VMEM((CHUNK, d), jnp.float32)] * NBUF,
            pltpu.VMEM_SHARED((N_PAD, d), jnp.float32),
            [pltpu.SemaphoreType.DMA] * NIDX,
            [pltpu.SemaphoreType.DMA] * NBUF,
        ],
    )
    def agg(z_hbm, edges_hbm, out_hbm,
            sidx, didx, rows, acc_sh, isems, gsems):
        cid = lax.axis_index("c")
        sid = lax.axis_index("s")
        r0 = sid * ROWS_PER_TILE
        wid = sid * NC + cid
        row0 = wid * BASE

        def idxload(c, q):
            pltpu.async_copy(edges_hbm.at[0, pl.ds(row0 + c, 1)],
                             sidx[q], isems[q])
            pltpu.async_copy(edges_hbm.at[1, pl.ds(row0 + c, 1)],
                             didx[q], isems[q])

        def idxwait(q):
            pltpu.make_async_copy(edges_hbm.at[0, pl.ds(0, 1)],
                                  sidx[q], isems[q]).wait()
            pltpu.make_async_copy(edges_hbm.at[1, pl.ds(0, 1)],
                                  didx[q], isems[q]).wait()

        def gather(q, b):
            pltpu.async_copy(z_hbm.at[sidx[q].at[0]], rows[b], gsems[b])

        def gwait(q, b):
            pltpu.make_async_copy(
                z_hbm.at[sidx[q].at[0]], rows[b], gsems[b]).wait()

        def scatter(q, b):
            pltpu.sync_copy(rows[b], acc_sh.at[didx[q].at[0]], add=True)

        zero16 = jnp.zeros((16,), jnp.float32)

        def zrow(j, car):
            for k in range(d // 16):
                rows[0][j, pl.ds(k * 16, 16)] = zero16
            return car

        lax.fori_loop(0, CHUNK, zrow, 0)
        full, rem = divmod(ROWS_PER_TILE, CHUNK)
        for t in range(full):
            pltpu.sync_copy(rows[0], acc_sh.at[pl.ds(r0 + t * CHUNK, CHUNK)])
        if rem:
            pltpu.sync_copy(rows[0].at[pl.ds(0, rem)],
                            acc_sh.at[pl.ds(r0 + full * CHUNK, rem)])
        plsc.subcore_barrier()

        for q in range(NIDX):
            idxload(q, q)
        for b in range(NBUF):
            idxwait(b)
            gather(b, b)

        def body(i, carry):
            cq = i * 4
            for q in range(4):
                b = q % 2
                c = cq + q
                gwait(q, b)
                scatter(q, b)

                @pl.when(c + 4 < BASE)
                def _():
                    idxload(c + 4, q)

                @pl.when(c + 2 < BASE)
                def _():
                    idxwait((q + 2) % 4)
                    gather((q + 2) % 4, b)

            return carry

        lax.fori_loop(0, QUADS, body, 0)
        for b in range(2):
            gwait(b, b)
            scatter(b, b)
        @pl.when(wid < EXTRA)
        def _():
            r = NW * BASE + wid - row0
            idxload(r, 0)
            idxwait(0)
            gather(0, 0)
            gwait(0, 0)
            scatter(0, 0)

        plsc.subcore_barrier()
        pltpu.sync_copy(acc_sh.at[pl.ds(r0, ROWS_PER_TILE)],
                        out_hbm.at[cid, pl.ds(r0, ROWS_PER_TILE)])

    return agg


def _make_deg_kernel():
    @functools.partial(
        pl.kernel,
        mesh=_MESH,
        out_type=jax.ShapeDtypeStruct((NC, N_PAD, 16), jnp.float32),
        compiler_params=_SC_PARAMS,
        scratch_types=[
            [pltpu.VMEM((1, CHUNK), jnp.int32)] * NIDX,
            pltpu.VMEM((CHUNK, 16), jnp.float32),
            pltpu.VMEM_SHARED((N_PAD, 16), jnp.float32),
            [pltpu.SemaphoreType.DMA] * NIDX,
        ],
    )
    def deg(edges_hbm, out_hbm, didx, ones_v, acc_sh, isems):
        cid = lax.axis_index("c")
        sid = lax.axis_index("s")
        r0 = sid * ROWS_PER_TILE
        wid = sid * NC + cid
        row0 = wid * BASE

        def idxload(c, q):
            pltpu.async_copy(edges_hbm.at[1, pl.ds(row0 + c, 1)],
                             didx[q], isems[q])

        def idxwait(q):
            pltpu.make_async_copy(edges_hbm.at[1, pl.ds(0, 1)],
                                  didx[q], isems[q]).wait()

        def scatter(q):
            pltpu.sync_copy(ones_v, acc_sh.at[didx[q].at[0]], add=True)

        zero16 = jnp.zeros((16,), jnp.float32)

        def zrow(j, car):
            ones_v[j, :] = zero16
            return car

        lax.fori_loop(0, CHUNK, zrow, 0)
        full, rem = divmod(ROWS_PER_TILE, CHUNK)
        for t in range(full):
            pltpu.sync_copy(ones_v, acc_sh.at[pl.ds(r0 + t * CHUNK, CHUNK)])
        if rem:
            pltpu.sync_copy(ones_v.at[pl.ds(0, rem)],
                            acc_sh.at[pl.ds(r0 + full * CHUNK, rem)])

        def orow(j, car):
            ones_v[j, :] = jnp.ones((16,), jnp.float32)
            return car

        lax.fori_loop(0, CHUNK, orow, 0)
        plsc.subcore_barrier()

        for q in range(NIDX):
            idxload(q, q)

        def body(i, carry):
            cq = i * 4
            for q in range(4):
                c = cq + q
                idxwait(q)
                scatter(q)

                @pl.when(c + 4 < BASE)
                def _():
                    idxload(c + 4, q)

            return carry

        lax.fori_loop(0, QUADS, body, 0)
        for q in range(2):
            idxwait(q)
            scatter(q)

        @pl.when(wid < EXTRA)
        def _():
            r = NW * BASE + wid - row0
            idxload(r, 0)
            idxwait(0)
            scatter(0)

        plsc.subcore_barrier()
        pltpu.sync_copy(acc_sh.at[pl.ds(r0, ROWS_PER_TILE)],
                        out_hbm.at[cid, pl.ds(r0, ROWS_PER_TILE)])

    return deg


_R = 1000


def _tc_mm(x, w1):
    def body(x_ref, w_ref, u_ref):
        u_ref[...] = jnp.dot(
            x_ref[...], w_ref[...], preferred_element_type=jnp.float32)

    return pl.pallas_call(
        body,
        grid=(N // _R,),
        in_specs=[
            pl.BlockSpec((_R, D_IN), lambda i: (i, 0)),
            pl.BlockSpec((D_IN, D_HID), lambda i: (0, 0)),
        ],
        out_specs=pl.BlockSpec((_R, D_HID), lambda i: (i, 0)),
        out_shape=jax.ShapeDtypeStruct((N, D_HID), jnp.float32),
    )(x, w1)


def _tc_scale(u1, degp):
    def body(u_ref, p_ref, z_ref, dis_ref):
        deg = 1.0 + p_ref[0] + p_ref[1]
        disv = lax.rsqrt(deg)
        dis_ref[...] = disv
        z_ref[...] = disv[:, 0:1] * u_ref[...]

    return pl.pallas_call(
        body,
        grid=(N // _R,),
        in_specs=[
            pl.BlockSpec((_R, D_HID), lambda i: (i, 0)),
            pl.BlockSpec((NC, _R, 16), lambda i: (0, i, 0)),
        ],
        out_specs=[
            pl.BlockSpec((_R, D_HID), lambda i: (i, 0)),
            pl.BlockSpec((_R, 16), lambda i: (i, 0)),
        ],
        out_shape=[
            jax.ShapeDtypeStruct((N, D_HID), jnp.float32),
            jax.ShapeDtypeStruct((N, 16), jnp.float32),
        ],
    )(u1, degp)


def _tc2(aggp, z1, dis, b1, w2):
    def body(a_ref, z_ref, dis_ref, b1_ref, w_ref, out_ref):
        pre = dis_ref[:, 0:1] * (a_ref[0] + a_ref[1] + z_ref[...]) + b1_ref[...]
        h = 1.0 / (1.0 + jnp.exp(-pre))
        mm = jnp.dot(h, w_ref[...], preferred_element_type=jnp.float32)
        out_ref[...] = dis_ref[:, 0:1] * mm

    return pl.pallas_call(
        body,
        grid=(N // _R,),
        in_specs=[
            pl.BlockSpec((NC, _R, D_HID), lambda i: (0, i, 0)),
            pl.BlockSpec((_R, D_HID), lambda i: (i, 0)),
            pl.BlockSpec((_R, 16), lambda i: (i, 0)),
            pl.BlockSpec((1, D_HID), lambda i: (0, 0)),
            pl.BlockSpec((D_HID, D_OUT), lambda i: (0, 0)),
        ],
        out_specs=pl.BlockSpec((_R, D_OUT), lambda i: (i, 0)),
        out_shape=jax.ShapeDtypeStruct((N, D_OUT), jnp.float32),
    )(aggp, z1, dis, b1, w2)


def _tc3(aggp, z2, dis, b2):
    def body(a_ref, z_ref, dis_ref, b2_ref, out_ref):
        out_ref[...] = (
            dis_ref[:, 0:1] * (a_ref[0] + a_ref[1] + z_ref[...]) + b2_ref[...]
        )

    return pl.pallas_call(
        body,
        grid=(N // _R,),
        in_specs=[
            pl.BlockSpec((NC, _R, D_OUT), lambda i: (0, i, 0)),
            pl.BlockSpec((_R, D_OUT), lambda i: (i, 0)),
            pl.BlockSpec((_R, 16), lambda i: (i, 0)),
            pl.BlockSpec((1, D_OUT), lambda i: (0, 0)),
        ],
        out_specs=pl.BlockSpec((_R, D_OUT), lambda i: (i, 0)),
        out_shape=jax.ShapeDtypeStruct((N, D_OUT), jnp.float32),
    )(aggp, z2, dis, b2)


_deg_kernel = _make_deg_kernel()
_agg128 = _make_agg_kernel(D_HID)
_agg64 = _make_agg_kernel(D_OUT)


def kernel(x, edges, W1, b1, W2, b2):
    edges3 = edges.astype(jnp.int32).reshape(2, N_CROWS, CHUNK)
    b1f = b1.reshape(1, D_HID)
    b2f = b2.reshape(1, D_OUT)

    degp = _deg_kernel(edges3)
    u1 = _tc_mm(x, W1)
    z1, dis = _tc_scale(u1, degp)
    agg1 = _agg128(z1, edges3)
    z2 = _tc2(agg1, z1, dis, b1f, W2)
    agg2 = _agg64(z2, edges3)
    out = _tc3(agg2, z2, dis, b2f)
    return out


# speedup vs baseline: 1.3119x; 1.0124x over previous
import functools

import jax
import jax.numpy as jnp
from jax import lax
from jax.experimental import pallas as pl
from jax.experimental.pallas import tpu as pltpu
from jax.experimental.pallas import tpu_sc as plsc

N = 10000
E = 320000
D_IN = 128
D_HID = 128
D_OUT = 64

NC = 2
NS = 16
NW = NC * NS

N_PAD = 10000
ROWS_PER_TILE = N_PAD // NS
E_PER_TILE = E // NW

CHUNK = 128
N_CROWS = E // CHUNK
BASE = N_CROWS // NW
EXTRA = N_CROWS - BASE * NW
QUADS = BASE // 4
assert BASE - QUADS * 4 == 2

_MESH = plsc.VectorSubcoreMesh(core_axis_name="c", subcore_axis_name="s")
_SC_PARAMS = pltpu.CompilerParams(use_tc_tiling_on_sc=False)

NBUF = 2
NIDX = 4


def _make_agg_kernel(d):
    @functools.partial(
        pl.kernel,
        mesh=_MESH,
        out_type=jax.ShapeDtypeStruct((NC, N_PAD, d), jnp.float32),
        compiler_params=_SC_PARAMS,
        scratch_types=[
            [pltpu.VMEM((1, CHUNK), jnp.int32)] * NIDX,
            [pltpu.VMEM((1, CHUNK), jnp.int32)] * NIDX,
            [pltpu.VMEM((CHUNK, d), jnp.float32)] * NBUF,
            pltpu.VMEM_SHARED((N_PAD, d), jnp.float32),
            [pltpu.SemaphoreType.DMA] * NIDX,
            [pltpu.SemaphoreType.DMA] * NBUF,
        ],
    )
    def agg(z_hbm, edges_hbm, out_hbm,
            sidx, didx, rows, acc_sh, isems, gsems):
        cid = lax.axis_index("c")
        sid = lax.axis_index("s")
        r0 = sid * ROWS_PER_TILE
        wid = sid * NC + cid
        row0 = wid * BASE

        def idxload(c, q):
            pltpu.async_copy(edges_hbm.at[0, pl.ds(row0 + c, 1)],
                             sidx[q], isems[q])
            pltpu.async_copy(edges_hbm.at[1, pl.ds(row0 + c, 1)],
                             didx[q], isems[q])

        def idxwait(q):
            pltpu.make_async_copy(edges_hbm.at[0, pl.ds(0, 1)],
                                  sidx[q], isems[q]).wait()
            pltpu.make_async_copy(edges_hbm.at[1, pl.ds(0, 1)],
                                  didx[q], isems[q]).wait()

        def gather(q, b):
            pltpu.async_copy(z_hbm.at[sidx[q].at[0]], rows[b], gsems[b])

        def gwait(q, b):
            pltpu.make_async_copy(
                z_hbm.at[sidx[q].at[0]], rows[b], gsems[b]).wait()

        def scatter(q, b):
            pltpu.sync_copy(rows[b], acc_sh.at[didx[q].at[0]], add=True)

        zero16 = jnp.zeros((16,), jnp.float32)

        def zrow(j, car):
            for k in range(d // 16):
                rows[0][j, pl.ds(k * 16, 16)] = zero16
            return car

        lax.fori_loop(0, CHUNK, zrow, 0)
        full, rem = divmod(ROWS_PER_TILE, CHUNK)
        for t in range(full):
            pltpu.sync_copy(rows[0], acc_sh.at[pl.ds(r0 + t * CHUNK, CHUNK)])
        if rem:
            pltpu.sync_copy(rows[0].at[pl.ds(0, rem)],
                            acc_sh.at[pl.ds(r0 + full * CHUNK, rem)])
        plsc.subcore_barrier()

        for q in range(NIDX):
            idxload(q, q)
        for b in range(NBUF):
            idxwait(b)
            gather(b, b)

        def body(i, carry):
            cq = i * 4
            for q in range(4):
                b = q % 2
                c = cq + q
                gwait(q, b)
                scatter(q, b)

                @pl.when(c + 4 < BASE)
                def _():
                    idxload(c + 4, q)

                @pl.when(c + 2 < BASE)
                def _():
                    idxwait((q + 2) % 4)
                    gather((q + 2) % 4, b)

            return carry

        lax.fori_loop(0, QUADS, body, 0)
        for b in range(2):
            gwait(b, b)
            scatter(b, b)
        @pl.when(wid < EXTRA)
        def _():
            r = NW * BASE + wid - row0
            idxload(r, 0)
            idxwait(0)
            gather(0, 0)
            gwait(0, 0)
            scatter(0, 0)

        plsc.subcore_barrier()
        pltpu.sync_copy(acc_sh.at[pl.ds(r0, ROWS_PER_TILE)],
                        out_hbm.at[cid, pl.ds(r0, ROWS_PER_TILE)])

    return agg


def _make_deg_kernel():
    @functools.partial(
        pl.kernel,
        mesh=_MESH,
        out_type=jax.ShapeDtypeStruct((NC, N_PAD, 16), jnp.float32),
        compiler_params=_SC_PARAMS,
        scratch_types=[
            pltpu.VMEM((BASE, CHUNK), jnp.int32),
            pltpu.VMEM((1, CHUNK), jnp.int32),
            pltpu.VMEM((CHUNK, 16), jnp.float32),
            pltpu.VMEM_SHARED((N_PAD, 16), jnp.float32),
            pltpu.SemaphoreType.DMA,
        ],
    )
    def deg(edges_hbm, out_hbm, dst_all, didx1, ones_v, acc_sh, ssem):
        cid = lax.axis_index("c")
        sid = lax.axis_index("s")
        r0 = sid * ROWS_PER_TILE
        wid = sid * NC + cid
        row0 = wid * BASE
        pltpu.sync_copy(edges_hbm.at[1, pl.ds(row0, BASE)], dst_all)

        zero16 = jnp.zeros((16,), jnp.float32)

        def zrow(j, car):
            ones_v[j, :] = zero16
            return car

        lax.fori_loop(0, CHUNK, zrow, 0)
        full, rem = divmod(ROWS_PER_TILE, CHUNK)
        for t in range(full):
            pltpu.sync_copy(ones_v, acc_sh.at[pl.ds(r0 + t * CHUNK, CHUNK)])
        if rem:
            pltpu.sync_copy(ones_v.at[pl.ds(0, rem)],
                            acc_sh.at[pl.ds(r0 + full * CHUNK, rem)])

        def orow(j, car):
            ones_v[j, :] = jnp.ones((16,), jnp.float32)
            return car

        lax.fori_loop(0, CHUNK, orow, 0)
        plsc.subcore_barrier()

        def body(i, carry):
            cb = i * 2
            for b in range(2):
                pltpu.async_copy(
                    ones_v, acc_sh.at[dst_all.at[cb + b]], ssem, add=True)
            for b in range(2):
                pltpu.make_async_copy(
                    ones_v, acc_sh.at[dst_all.at[cb + b]], ssem).wait()
            return carry

        lax.fori_loop(0, BASE // 2, body, 0)

        @pl.when(wid < EXTRA)
        def _():
            pltpu.async_copy(edges_hbm.at[1, pl.ds(NW * BASE + wid, 1)],
                             didx1, ssem)
            pltpu.make_async_copy(edges_hbm.at[1, pl.ds(0, 1)],
                                  didx1, ssem).wait()
            pltpu.sync_copy(ones_v, acc_sh.at[didx1.at[0]], add=True)

        plsc.subcore_barrier()
        pltpu.sync_copy(acc_sh.at[pl.ds(r0, ROWS_PER_TILE)],
                        out_hbm.at[cid, pl.ds(r0, ROWS_PER_TILE)])

    return deg


_R = 1000


def _tc_mm(x, w1):
    def body(x_ref, w_ref, u_ref):
        u_ref[...] = jnp.dot(
            x_ref[...], w_ref[...], preferred_element_type=jnp.float32)

    return pl.pallas_call(
        body,
        grid=(N // _R,),
        in_specs=[
            pl.BlockSpec((_R, D_IN), lambda i: (i, 0)),
            pl.BlockSpec((D_IN, D_HID), lambda i: (0, 0)),
        ],
        out_specs=pl.BlockSpec((_R, D_HID), lambda i: (i, 0)),
        out_shape=jax.ShapeDtypeStruct((N, D_HID), jnp.float32),
    )(x, w1)


def _tc_scale(u1, degp):
    def body(u_ref, p_ref, z_ref, dis_ref):
        deg = 1.0 + p_ref[0] + p_ref[1]
        disv = lax.rsqrt(deg)
        dis_ref[...] = disv
        z_ref[...] = disv[:, 0:1] * u_ref[...]

    return pl.pallas_call(
        body,
        grid=(N // _R,),
        in_specs=[
            pl.BlockSpec((_R, D_HID), lambda i: (i, 0)),
            pl.BlockSpec((NC, _R, 16), lambda i: (0, i, 0)),
        ],
        out_specs=[
            pl.BlockSpec((_R, D_HID), lambda i: (i, 0)),
            pl.BlockSpec((_R, 16), lambda i: (i, 0)),
        ],
        out_shape=[
            jax.ShapeDtypeStruct((N, D_HID), jnp.float32),
            jax.ShapeDtypeStruct((N, 16), jnp.float32),
        ],
    )(u1, degp)


def _tc2(aggp, z1, dis, b1, w2):
    def body(a_ref, z_ref, dis_ref, b1_ref, w_ref, out_ref):
        pre = dis_ref[:, 0:1] * (a_ref[0] + a_ref[1] + z_ref[...]) + b1_ref[...]
        h = 1.0 / (1.0 + jnp.exp(-pre))
        mm = jnp.dot(h, w_ref[...], preferred_element_type=jnp.float32)
        out_ref[...] = dis_ref[:, 0:1] * mm

    return pl.pallas_call(
        body,
        grid=(N // _R,),
        in_specs=[
            pl.BlockSpec((NC, _R, D_HID), lambda i: (0, i, 0)),
            pl.BlockSpec((_R, D_HID), lambda i: (i, 0)),
            pl.BlockSpec((_R, 16), lambda i: (i, 0)),
            pl.BlockSpec((1, D_HID), lambda i: (0, 0)),
            pl.BlockSpec((D_HID, D_OUT), lambda i: (0, 0)),
        ],
        out_specs=pl.BlockSpec((_R, D_OUT), lambda i: (i, 0)),
        out_shape=jax.ShapeDtypeStruct((N, D_OUT), jnp.float32),
    )(aggp, z1, dis, b1, w2)


def _tc3(aggp, z2, dis, b2):
    def body(a_ref, z_ref, dis_ref, b2_ref, out_ref):
        out_ref[...] = (
            dis_ref[:, 0:1] * (a_ref[0] + a_ref[1] + z_ref[...]) + b2_ref[...]
        )

    return pl.pallas_call(
        body,
        grid=(N // _R,),
        in_specs=[
            pl.BlockSpec((NC, _R, D_OUT), lambda i: (0, i, 0)),
            pl.BlockSpec((_R, D_OUT), lambda i: (i, 0)),
            pl.BlockSpec((_R, 16), lambda i: (i, 0)),
            pl.BlockSpec((1, D_OUT), lambda i: (0, 0)),
        ],
        out_specs=pl.BlockSpec((_R, D_OUT), lambda i: (i, 0)),
        out_shape=jax.ShapeDtypeStruct((N, D_OUT), jnp.float32),
    )(aggp, z2, dis, b2)


_deg_kernel = _make_deg_kernel()
_agg128 = _make_agg_kernel(D_HID)
_agg64 = _make_agg_kernel(D_OUT)


def kernel(x, edges, W1, b1, W2, b2):
    edges3 = edges.astype(jnp.int32).reshape(2, N_CROWS, CHUNK)
    b1f = b1.reshape(1, D_HID)
    b2f = b2.reshape(1, D_OUT)

    degp = _deg_kernel(edges3)
    u1 = _tc_mm(x, W1)
    z1, dis = _tc_scale(u1, degp)
    agg1 = _agg128(z1, edges3)
    z2 = _tc2(agg1, z1, dis, b1f, W2)
    agg2 = _agg64(z2, edges3)
    out = _tc3(agg2, z2, dis, b2f)
    return out
